# baseline (speedup 1.0000x reference)
import functools

import jax
import jax.numpy as jnp
from jax import lax
from jax.experimental import pallas as pl
from jax.experimental.pallas import tpu as pltpu
from jax.experimental.pallas import tpu_sc as plsc

_N = 10000
_E = 320000
_D = 128
_L = 3
_EPS = 1e-5

_NC, _NS = 2, 16
_NW = _NC * _NS
_PARTS = (86016, 233984)
_CH = 128
_KG = 3
_NPT = 632
_NPAD = _NS * _NPT

_ET = 8000
_NT = 10000


def _ln(x, g, b):
    m = jnp.mean(x, axis=-1, keepdims=True)
    v = jnp.mean(jnp.square(x - m), axis=-1, keepdims=True)
    return (x - m) * lax.rsqrt(v + _EPS) * g + b


def _pack_bf16(x):
    b = jax.lax.bitcast_convert_type(x.astype(jnp.bfloat16),
                                     jnp.uint16).astype(jnp.int32)
    return b[:, :64] | (b[:, 64:] << 16)


def _unpack_bf16(w):
    lo = jax.lax.bitcast_convert_type(w << 16, jnp.float32)
    hi = jax.lax.bitcast_convert_type(w & jnp.int32(-65536), jnp.float32)
    return jnp.concatenate([lo, hi], axis=1)


def _mm_body(x_ref, w_ref, o_ref):
    o_ref[...] = jnp.dot(x_ref[...], w_ref[...],
                         preferred_element_type=jnp.float32)


def _rows_matmul(x, w):
    n, k = x.shape
    return pl.pallas_call(
        _mm_body,
        grid=(n // _NT,),
        in_specs=[pl.BlockSpec((_NT, k), lambda i: (i, 0)),
                  pl.BlockSpec(w.shape, lambda i: (0, 0))],
        out_specs=pl.BlockSpec((_NT, w.shape[1]), lambda i: (i, 0)),
        out_shape=jax.ShapeDtypeStruct((n, w.shape[1]), jnp.float32),
    )(x, w)


def _edge_body(g_ref, bx_ref, w1_ref, b1_ref, g1_ref, B1_ref,
               w2_ref, b2_ref, g2_ref, B2_ref, o_ref):
    u = (g_ref[...]
         + jnp.dot(bx_ref[...], w1_ref[...], preferred_element_type=jnp.float32)
         + b1_ref[...])
    u = jnp.maximum(_ln(u, g1_ref[...], B1_ref[...]), 0.0)
    t = jnp.dot(u, w2_ref[...], preferred_element_type=jnp.float32) + b2_ref[...]
    o_ref[...] = jnp.maximum(_ln(t, g2_ref[...], B2_ref[...]), 0.0)


def _edge_mlp(gath, bond, w1b, b1, g1, B1, w2, b2, g2, B2):
    size = gath.shape[0]
    et = size // 16
    assert et % 8 == 0 and et * 16 == size
    row = pl.BlockSpec((et, _D), lambda i: (i, 0))

    def full(a):
        return pl.BlockSpec(a.shape, lambda i: (0,) * a.ndim)

    return pl.pallas_call(
        _edge_body,
        grid=(16,),
        in_specs=[row, row, full(w1b), full(b1), full(g1), full(B1),
                  full(w2), full(b2), full(g2), full(B2)],
        out_specs=row,
        out_shape=jax.ShapeDtypeStruct((size, _D), jnp.float32),
    )(gath, bond, w1b, b1, g1, B1, w2, b2, g2, B2)


def _node_body(ax_ref, p0_ref, p1_ref, w1a_ref, w1b_ref, b1_ref, g1_ref,
               B1_ref, w2_ref, b2_ref, g2_ref, B2_ref, wn_ref,
               h_ref, hp_ref):
    agg = p0_ref[...] + p1_ref[...]
    t = (jnp.dot(ax_ref[...], w1a_ref[...], preferred_element_type=jnp.float32)
         + jnp.dot(agg, w1b_ref[...], preferred_element_type=jnp.float32)
         + b1_ref[...])
    t = jnp.maximum(_ln(t, g1_ref[...], B1_ref[...]), 0.0)
    t2 = jnp.dot(t, w2_ref[...], preferred_element_type=jnp.float32) + b2_ref[...]
    h = jnp.maximum(_ln(t2, g2_ref[...], B2_ref[...]), 0.0)
    h_ref[...] = h
    hp_ref[...] = jnp.dot(h, wn_ref[...], preferred_element_type=jnp.float32)


def _node_mlp(atom, part0, part1, w1a, w1b, b1, g1, B1, w2, b2, g2, B2,
              wnext):
    grid = _N // _NT
    row = pl.BlockSpec((_NT, _D), lambda i: (i, 0))
    rowp = pl.BlockSpec((_NT, _D // 2), lambda i: (i, 0))

    def full(a):
        return pl.BlockSpec(a.shape, lambda i: (0,) * a.ndim)

    return pl.pallas_call(
        _node_body,
        grid=(grid,),
        in_specs=[row, row, row, full(w1a), full(w1b), full(b1), full(g1),
                  full(B1), full(w2), full(b2), full(g2), full(B2),
                  full(wnext)],
        out_specs=(row, row),
        out_shape=(jax.ShapeDtypeStruct((_N, _D), jnp.float32),
                   jax.ShapeDtypeStruct((_N, _D), jnp.float32)),
    )(atom, part0, part1, w1a, w1b, b1, g1, B1, w2, b2, g2, B2, wnext)


def _sc_gather(hp, src):
    size = src.shape[0]
    epw = size // _NW
    nfull = epw // _CH
    tail = epw - nfull * _CH
    nrg = nfull // _KG
    assert nrg * _KG == nfull and epw % 8 == 0
    mesh = plsc.VectorSubcoreMesh(core_axis_name="c", subcore_axis_name="s")

    @functools.partial(
        pl.kernel, mesh=mesh,
        out_type=jax.ShapeDtypeStruct((size, _D), jnp.float32),
        scratch_types=[pltpu.VMEM((epw,), jnp.int32),
                       pltpu.VMEM((2, _KG * _CH, _D), jnp.float32),
                       pltpu.VMEM((max(tail, 8), _D), jnp.float32),
                       pltpu.SemaphoreType.DMA,
                       pltpu.SemaphoreType.DMA],
    )
    def gather_k(hp_hbm, src_hbm, out_hbm, idx_all, rows_v, rowst_v,
                 semg, semo):
        w = lax.axis_index("c") * _NS + lax.axis_index("s")
        base = w * epw
        pltpu.sync_copy(src_hbm.at[pl.ds(base, epw)], idx_all)

        gs = [None, None]
        outs = [None, None]

        def start_gathers(r):
            a = r % 2
            gs[a] = [pltpu.async_copy(
                hp_hbm.at[idx_all.at[pl.ds((r * _KG + k) * _CH, _CH)]],
                rows_v.at[a, pl.ds(k * _CH, _CH)], semg)
                for k in range(_KG)]

        start_gathers(0)
        for r in range(nrg):
            a = r % 2
            if r + 1 < nrg:
                if outs[1 - a] is not None:
                    outs[1 - a].wait()
                start_gathers(r + 1)
            for d in gs[a]:
                d.wait()
            outs[a] = pltpu.async_copy(
                rows_v.at[a],
                out_hbm.at[pl.ds(base + r * _KG * _CH, _KG * _CH)], semo)
        for o in outs:
            if o is not None:
                o.wait()
        if tail:
            off = base + nfull * _CH
            pltpu.async_copy(hp_hbm.at[idx_all.at[pl.ds(nfull * _CH, tail)]],
                             rowst_v, semg).wait()
            pltpu.sync_copy(rowst_v, out_hbm.at[pl.ds(off, tail)])

    return gather_k(hp, src)


def _sc_scatter(bh, dst, init0, init1):
    size = dst.shape[0]
    epw = size // _NW
    nfull = epw // _CH
    tail = epw - nfull * _CH
    assert epw % 8 == 0
    mesh = plsc.VectorSubcoreMesh(core_axis_name="c", subcore_axis_name="s")

    @functools.partial(
        pl.kernel, mesh=mesh,
        out_type=(jax.ShapeDtypeStruct((_NPAD, _D), jnp.float32),
                  jax.ShapeDtypeStruct((_NPAD, _D), jnp.float32)),
        scratch_types=[pltpu.VMEM_SHARED((_NPAD, _D), jnp.float32),
                       pltpu.VMEM((2, 1, _CH), jnp.int32),
                       pltpu.VMEM((2, _CH, _D), jnp.float32),
                       pltpu.VMEM((max(tail, 8),), jnp.int32),
                       pltpu.VMEM((max(tail, 8), _D), jnp.float32),
                       pltpu.SemaphoreType.DMA,
                       pltpu.SemaphoreType.DMA],
    )
    def scatter_k(bh_hbm, dst_hbm, i0_hbm, i1_hbm, out0_hbm, out1_hbm, acc,
                  idx_v, rows_v, idxt_v, rowst_v, seml, sema):
        c = lax.axis_index("c")
        s = lax.axis_index("s")

        @pl.when(c == 0)
        def _():
            pltpu.sync_copy(i0_hbm.at[pl.ds(s * _NPT, _NPT)],
                            acc.at[pl.ds(s * _NPT, _NPT)])

        @pl.when(c == 1)
        def _():
            pltpu.sync_copy(i1_hbm.at[pl.ds(s * _NPT, _NPT)],
                            acc.at[pl.ds(s * _NPT, _NPT)])

        plsc.subcore_barrier()
        base = (c * _NS + s) * epw

        loads = [None, None]
        adds = [None, None]

        def start_round(r):
            a = r % 2
            loads[a] = [
                pltpu.async_copy(bh_hbm.at[pl.ds(base + r * _CH, _CH)],
                                 rows_v.at[a], seml),
                pltpu.async_copy(dst_hbm.at[pl.ds(base + r * _CH, _CH)],
                                 idx_v.at[a, 0], seml)]

        def add_round(r):
            a = r % 2
            adds[a] = [pltpu.async_copy(rows_v.at[a],
                                        acc.at[idx_v.at[a, 0]], sema,
                                        add=True)]

        start_round(0)
        for r in range(nfull):
            a = r % 2
            if r + 1 < nfull:
                if adds[1 - a] is not None:
                    for d in adds[1 - a]:
                        d.wait()
                    adds[1 - a] = None
                start_round(r + 1)
            for d in loads[a]:
                d.wait()
            add_round(r)
        for ds in adds:
            if ds is not None:
                for d in ds:
                    d.wait()
        if tail:
            off = base + nfull * _CH
            pltpu.sync_copy(dst_hbm.at[pl.ds(off, tail)], idxt_v)
            pltpu.sync_copy(bh_hbm.at[pl.ds(off, tail)], rowst_v)
            pltpu.sync_copy(rowst_v, acc.at[idxt_v], add=True)
        plsc.subcore_barrier()

        @pl.when(c == 0)
        def _():
            pltpu.sync_copy(acc.at[pl.ds(s * _NPT, _NPT)],
                            out0_hbm.at[pl.ds(s * _NPT, _NPT)])

        @pl.when(c == 1)
        def _():
            pltpu.sync_copy(acc.at[pl.ds(s * _NPT, _NPT)],
                            out1_hbm.at[pl.ds(s * _NPT, _NPT)])

    return scatter_k(bh, dst, init0, init1)


def kernel(atom_x, bond_x, subtree_h, bW1, bb1, bg1, bB1, bW2, bb2, bg2,
           bB2, tW1, tb1, tg1, tB1, tW2, tb2, tg2, tB2, edge_index):
    src = edge_index[0]
    dst = edge_index[1]
    bounds = []
    o = 0
    for p in _PARTS:
        bounds.append((o, o + p))
        o += p
    src_p = [src[a:b] for a, b in bounds]
    dst_p = [dst[a:b] for a, b in bounds]
    bond_p = [bond_x[a:b].astype(jnp.bfloat16) for a, b in bounds]
    zeros_init = jnp.zeros((_NPAD, _D), jnp.float32)

    hp = _rows_matmul(subtree_h, bW1[0, :_D, :])
    h = subtree_h
    for l in range(_L):
        ew = (bW1[l, _D:, :].astype(jnp.bfloat16), bb1[l][None],
              bg1[l][None], bB1[l][None], bW2[l], bb2[l][None],
              bg2[l][None], bB2[l][None])
        p0, p1 = zeros_init, zeros_init
        for i in range(len(_PARTS)):
            g_i = _sc_gather(hp, src_p[i])
            bh_i = _edge_mlp(g_i, bond_p[i], *ew)
            p0, p1 = _sc_scatter(bh_i, dst_p[i], p0, p1)
        wnext = bW1[(l + 1) % _L, :_D, :]
        h, hp = _node_mlp(atom_x, p0, p1, tW1[l, :_D, :], tW1[l, _D:, :],
                          tb1[l][None], tg1[l][None], tB1[l][None], tW2[l],
                          tb2[l][None], tg2[l][None], tB2[l][None], wnext)
    return h


# speedup vs baseline: 1.0486x; 1.0486x over previous
import functools

import jax
import jax.numpy as jnp
from jax import lax
from jax.experimental import pallas as pl
from jax.experimental.pallas import tpu as pltpu
from jax.experimental.pallas import tpu_sc as plsc

_N = 10000
_E = 320000
_D = 128
_L = 3
_EPS = 1e-5

_NC, _NS = 2, 16
_NW = _NC * _NS
_PARTS = (160000, 160000)
_CH = 128
_KG = 3
_NPT = 632
_NPAD = _NS * _NPT

_ET = 8000
_NT = 10000


def _ln(x, g, b):
    m = jnp.mean(x, axis=-1, keepdims=True)
    v = jnp.mean(jnp.square(x - m), axis=-1, keepdims=True)
    return (x - m) * lax.rsqrt(v + _EPS) * g + b


def _pack_bf16(x):
    b = jax.lax.bitcast_convert_type(x.astype(jnp.bfloat16),
                                     jnp.uint16).astype(jnp.int32)
    return b[:, :64] | (b[:, 64:] << 16)


def _unpack_bf16(w):
    lo = jax.lax.bitcast_convert_type(w << 16, jnp.float32)
    hi = jax.lax.bitcast_convert_type(w & jnp.int32(-65536), jnp.float32)
    return jnp.concatenate([lo, hi], axis=1)


def _mm_body(x_ref, w_ref, o_ref):
    o_ref[...] = jnp.dot(x_ref[...], w_ref[...],
                         preferred_element_type=jnp.float32)


def _rows_matmul(x, w):
    n, k = x.shape
    return pl.pallas_call(
        _mm_body,
        grid=(n // _NT,),
        in_specs=[pl.BlockSpec((_NT, k), lambda i: (i, 0)),
                  pl.BlockSpec(w.shape, lambda i: (0, 0))],
        out_specs=pl.BlockSpec((_NT, w.shape[1]), lambda i: (i, 0)),
        out_shape=jax.ShapeDtypeStruct((n, w.shape[1]), jnp.float32),
    )(x, w)


def _edge_body(g_ref, bx_ref, w1_ref, b1_ref, g1_ref, B1_ref,
               w2_ref, b2_ref, g2_ref, B2_ref, o_ref):
    u = (g_ref[...]
         + jnp.dot(bx_ref[...], w1_ref[...], preferred_element_type=jnp.float32)
         + b1_ref[...])
    u = jnp.maximum(_ln(u, g1_ref[...], B1_ref[...]), 0.0)
    t = jnp.dot(u, w2_ref[...], preferred_element_type=jnp.float32) + b2_ref[...]
    o_ref[...] = jnp.maximum(_ln(t, g2_ref[...], B2_ref[...]), 0.0)


def _edge_mlp(gath, bond, w1b, b1, g1, B1, w2, b2, g2, B2):
    size = gath.shape[0]
    row = pl.BlockSpec((_ET, _D), lambda i: (i, 0))

    def full(a):
        return pl.BlockSpec(a.shape, lambda i: (0,) * a.ndim)

    return pl.pallas_call(
        _edge_body,
        grid=(size // _ET,),
        in_specs=[row, row, full(w1b), full(b1), full(g1), full(B1),
                  full(w2), full(b2), full(g2), full(B2)],
        out_specs=row,
        out_shape=jax.ShapeDtypeStruct((size, _D), jnp.float32),
    )(gath, bond, w1b, b1, g1, B1, w2, b2, g2, B2)


def _node_body(ax_ref, p0_ref, p1_ref, w1a_ref, w1b_ref, b1_ref, g1_ref,
               B1_ref, w2_ref, b2_ref, g2_ref, B2_ref, wn_ref,
               h_ref, hp_ref):
    agg = p0_ref[...] + p1_ref[...]
    t = (jnp.dot(ax_ref[...], w1a_ref[...], preferred_element_type=jnp.float32)
         + jnp.dot(agg, w1b_ref[...], preferred_element_type=jnp.float32)
         + b1_ref[...])
    t = jnp.maximum(_ln(t, g1_ref[...], B1_ref[...]), 0.0)
    t2 = jnp.dot(t, w2_ref[...], preferred_element_type=jnp.float32) + b2_ref[...]
    h = jnp.maximum(_ln(t2, g2_ref[...], B2_ref[...]), 0.0)
    h_ref[...] = h
    hp_ref[...] = jnp.dot(h, wn_ref[...], preferred_element_type=jnp.float32)


def _node_mlp(atom, part0, part1, w1a, w1b, b1, g1, B1, w2, b2, g2, B2,
              wnext):
    grid = _N // _NT
    row = pl.BlockSpec((_NT, _D), lambda i: (i, 0))
    rowp = pl.BlockSpec((_NT, _D // 2), lambda i: (i, 0))

    def full(a):
        return pl.BlockSpec(a.shape, lambda i: (0,) * a.ndim)

    return pl.pallas_call(
        _node_body,
        grid=(grid,),
        in_specs=[row, row, row, full(w1a), full(w1b), full(b1), full(g1),
                  full(B1), full(w2), full(b2), full(g2), full(B2),
                  full(wnext)],
        out_specs=(row, row),
        out_shape=(jax.ShapeDtypeStruct((_N, _D), jnp.float32),
                   jax.ShapeDtypeStruct((_N, _D), jnp.float32)),
    )(atom, part0, part1, w1a, w1b, b1, g1, B1, w2, b2, g2, B2, wnext)


def _sc_gather(hp, src):
    size = src.shape[0]
    epw = size // _NW
    nfull = epw // _CH
    tail = epw - nfull * _CH
    nrg = nfull // _KG
    assert nrg * _KG == nfull and epw % 8 == 0
    mesh = plsc.VectorSubcoreMesh(core_axis_name="c", subcore_axis_name="s")

    @functools.partial(
        pl.kernel, mesh=mesh,
        out_type=jax.ShapeDtypeStruct((size, _D), jnp.float32),
        scratch_types=[pltpu.VMEM((epw,), jnp.int32),
                       pltpu.VMEM((2, _KG * _CH, _D), jnp.float32),
                       pltpu.VMEM((max(tail, 8), _D), jnp.float32),
                       pltpu.SemaphoreType.DMA,
                       pltpu.SemaphoreType.DMA],
    )
    def gather_k(hp_hbm, src_hbm, out_hbm, idx_all, rows_v, rowst_v,
                 semg, semo):
        w = lax.axis_index("c") * _NS + lax.axis_index("s")
        base = w * epw
        pltpu.sync_copy(src_hbm.at[pl.ds(base, epw)], idx_all)

        gs = [None, None]
        outs = [None, None]

        def start_gathers(r):
            a = r % 2
            gs[a] = [pltpu.async_copy(
                hp_hbm.at[idx_all.at[pl.ds((r * _KG + k) * _CH, _CH)]],
                rows_v.at[a, pl.ds(k * _CH, _CH)], semg)
                for k in range(_KG)]

        start_gathers(0)
        for r in range(nrg):
            a = r % 2
            if r + 1 < nrg:
                if outs[1 - a] is not None:
                    outs[1 - a].wait()
                start_gathers(r + 1)
            for d in gs[a]:
                d.wait()
            outs[a] = pltpu.async_copy(
                rows_v.at[a],
                out_hbm.at[pl.ds(base + r * _KG * _CH, _KG * _CH)], semo)
        for o in outs:
            if o is not None:
                o.wait()
        if tail:
            off = base + nfull * _CH
            pltpu.async_copy(hp_hbm.at[idx_all.at[pl.ds(nfull * _CH, tail)]],
                             rowst_v, semg).wait()
            pltpu.sync_copy(rowst_v, out_hbm.at[pl.ds(off, tail)])

    return gather_k(hp, src)


def _sc_scatter(bh, dst, init0, init1):
    size = dst.shape[0]
    epw = size // _NW
    nfull = epw // _CH
    tail = epw - nfull * _CH
    assert epw % 8 == 0
    mesh = plsc.VectorSubcoreMesh(core_axis_name="c", subcore_axis_name="s")

    @functools.partial(
        pl.kernel, mesh=mesh,
        out_type=(jax.ShapeDtypeStruct((_NPAD, _D), jnp.float32),
                  jax.ShapeDtypeStruct((_NPAD, _D), jnp.float32)),
        scratch_types=[pltpu.VMEM_SHARED((_NPAD, _D), jnp.float32),
                       pltpu.VMEM((2, 1, _CH), jnp.int32),
                       pltpu.VMEM((2, _CH, _D), jnp.float32),
                       pltpu.VMEM((max(tail, 8),), jnp.int32),
                       pltpu.VMEM((max(tail, 8), _D), jnp.float32),
                       pltpu.SemaphoreType.DMA,
                       pltpu.SemaphoreType.DMA],
    )
    def scatter_k(bh_hbm, dst_hbm, i0_hbm, i1_hbm, out0_hbm, out1_hbm, acc,
                  idx_v, rows_v, idxt_v, rowst_v, seml, sema):
        c = lax.axis_index("c")
        s = lax.axis_index("s")

        @pl.when(c == 0)
        def _():
            pltpu.sync_copy(i0_hbm.at[pl.ds(s * _NPT, _NPT)],
                            acc.at[pl.ds(s * _NPT, _NPT)])

        @pl.when(c == 1)
        def _():
            pltpu.sync_copy(i1_hbm.at[pl.ds(s * _NPT, _NPT)],
                            acc.at[pl.ds(s * _NPT, _NPT)])

        plsc.subcore_barrier()
        base = (c * _NS + s) * epw

        loads = [None, None]
        adds = [None, None]

        def start_round(r):
            a = r % 2
            loads[a] = [
                pltpu.async_copy(bh_hbm.at[pl.ds(base + r * _CH, _CH)],
                                 rows_v.at[a], seml),
                pltpu.async_copy(dst_hbm.at[pl.ds(base + r * _CH, _CH)],
                                 idx_v.at[a, 0], seml)]

        def add_round(r):
            a = r % 2
            adds[a] = [pltpu.async_copy(rows_v.at[a],
                                        acc.at[idx_v.at[a, 0]], sema,
                                        add=True)]

        start_round(0)
        for r in range(nfull):
            a = r % 2
            if r + 1 < nfull:
                if adds[1 - a] is not None:
                    for d in adds[1 - a]:
                        d.wait()
                    adds[1 - a] = None
                start_round(r + 1)
            for d in loads[a]:
                d.wait()
            add_round(r)
        for ds in adds:
            if ds is not None:
                for d in ds:
                    d.wait()
        if tail:
            off = base + nfull * _CH
            pltpu.sync_copy(dst_hbm.at[pl.ds(off, tail)], idxt_v)
            pltpu.sync_copy(bh_hbm.at[pl.ds(off, tail)], rowst_v)
            pltpu.sync_copy(rowst_v, acc.at[idxt_v], add=True)
        plsc.subcore_barrier()

        @pl.when(c == 0)
        def _():
            pltpu.sync_copy(acc.at[pl.ds(s * _NPT, _NPT)],
                            out0_hbm.at[pl.ds(s * _NPT, _NPT)])

        @pl.when(c == 1)
        def _():
            pltpu.sync_copy(acc.at[pl.ds(s * _NPT, _NPT)],
                            out1_hbm.at[pl.ds(s * _NPT, _NPT)])

    return scatter_k(bh, dst, init0, init1)


def kernel(atom_x, bond_x, subtree_h, bW1, bb1, bg1, bB1, bW2, bb2, bg2,
           bB2, tW1, tb1, tg1, tB1, tW2, tb2, tg2, tB2, edge_index):
    src = edge_index[0]
    dst = edge_index[1]
    bounds = []
    o = 0
    for p in _PARTS:
        bounds.append((o, o + p))
        o += p
    src_p = [src[a:b] for a, b in bounds]
    dst_p = [dst[a:b] for a, b in bounds]
    bond_p = [bond_x[a:b].astype(jnp.bfloat16) for a, b in bounds]
    zeros_init = jnp.zeros((_NPAD, _D), jnp.float32)

    hp = _rows_matmul(subtree_h, bW1[0, :_D, :])
    h = subtree_h
    for l in range(_L):
        ew = (bW1[l, _D:, :].astype(jnp.bfloat16), bb1[l][None],
              bg1[l][None], bB1[l][None], bW2[l], bb2[l][None],
              bg2[l][None], bB2[l][None])
        p0, p1 = zeros_init, zeros_init
        for i in range(len(_PARTS)):
            g_i = _sc_gather(hp, src_p[i])
            bh_i = _edge_mlp(g_i, bond_p[i], *ew)
            p0, p1 = _sc_scatter(bh_i, dst_p[i], p0, p1)
        wnext = bW1[(l + 1) % _L, :_D, :]
        h, hp = _node_mlp(atom_x, p0, p1, tW1[l, :_D, :], tW1[l, _D:, :],
                          tb1[l][None], tg1[l][None], tB1[l][None], tW2[l],
                          tb2[l][None], tg2[l][None], tB2[l][None], wnext)
    return h
